# baseline (speedup 1.0000x reference)
import functools

import jax
import jax.numpy as jnp
from jax import lax
from jax.experimental import pallas as pl
from jax.experimental.pallas import tpu as pltpu
from jax.experimental.pallas import tpu_sc as plsc

NC = 2
NS = 16
GSZ = 512
FAST_CORE_SHARE = 0.600


def _mesh():
  return plsc.VectorSubcoreMesh(core_axis_name="c", subcore_axis_name="s")


def _make_deg_kernel(groups, rpt, np_rows):
  @functools.partial(
      pl.kernel,
      out_type=jax.ShapeDtypeStruct((NC, np_rows, 16), jnp.float32),
      mesh=_mesh(),
      compiler_params=pltpu.CompilerParams(use_tc_tiling_on_sc=False),
      scratch_types=[
          pltpu.VMEM((GSZ,), jnp.int32),
          pltpu.VMEM((GSZ,), jnp.int32),
          pltpu.VMEM((GSZ, 16), jnp.float32),
          pltpu.SemaphoreType.DMA,
          pltpu.SemaphoreType.DMA,
          pltpu.VMEM_SHARED((np_rows, 16), jnp.float32),
      ],
  )
  def deg_kernel(dst_hbm, ones_hbm, zeros_hbm, out_hbm,
                 dbuf0, dbuf1, ones_v, ssem0, ssem1, deg_sh):
    c = lax.axis_index("c")
    s = lax.axis_index("s")
    dbuf = (dbuf0, dbuf1)
    ssem = (ssem0, ssem1)
    pltpu.sync_copy(ones_hbm, ones_v)
    pltpu.sync_copy(zeros_hbm, deg_sh.at[pl.ds(s * rpt, rpt)])
    plsc.subcore_barrier()
    tb = (c * NS + s) * groups * GSZ

    def fire(gv, p):
      pltpu.sync_copy(dst_hbm.at[pl.ds(tb + gv * GSZ, GSZ)], dbuf[p])
      pltpu.async_copy(ones_v, deg_sh.at[dbuf[p]], ssem[p], add=True)

    def drain(p):
      pltpu.make_async_copy(ones_v, deg_sh.at[dbuf[p]], ssem[p]).wait()

    fire(0, 0)
    fire(1, 1)

    def body(k, carry):
      g = 2 + 2 * k
      drain(0)
      fire(g, 0)
      drain(1)
      fire(g + 1, 1)
      return carry

    lax.fori_loop(0, (groups - 2) // 2, body, 0)
    drain(0)
    drain(1)
    plsc.subcore_barrier()
    pltpu.sync_copy(deg_sh.at[pl.ds(s * rpt, rpt)],
                    out_hbm.at[c].at[pl.ds(s * rpt, rpt)])

  return deg_kernel


def _make_edge_kernel(g0, g1, rpt, np_rows):
  @functools.partial(
      pl.kernel,
      out_type=jax.ShapeDtypeStruct((NC, np_rows, 16), jnp.float32),
      mesh=_mesh(),
      compiler_params=pltpu.CompilerParams(use_tc_tiling_on_sc=False),
      scratch_types=[
          [pltpu.VMEM((GSZ,), jnp.int32)] * 4,
          [pltpu.VMEM((GSZ,), jnp.int32)] * 4,
          [pltpu.VMEM((GSZ, 16), jnp.float32)] * 2,
          [pltpu.SemaphoreType.DMA] * 2,
          [pltpu.SemaphoreType.DMA] * 2,
          pltpu.VMEM_SHARED((np_rows, 16), jnp.float32),
      ],
  )
  def edge_kernel(src_hbm, dst_hbm, u_hbm, zeros_hbm, out_hbm,
                  sbuf, dbuf, rows, gsem, ssem, agg_sh):
    c = lax.axis_index("c")
    s = lax.axis_index("s")
    pltpu.sync_copy(zeros_hbm, agg_sh.at[pl.ds(s * rpt, rpt)])
    plsc.subcore_barrier()
    groups = lax.select(c == 0, g0, g1)
    tb = lax.select(c == 0, s * g0, NS * g0 + s * g1) * GSZ

    def load_idx(gv, q):
      pltpu.sync_copy(src_hbm.at[pl.ds(tb + gv * GSZ, GSZ)], sbuf[q])
      pltpu.sync_copy(dst_hbm.at[pl.ds(tb + gv * GSZ, GSZ)], dbuf[q])

    def fire_gather(q, p):
      pltpu.async_copy(u_hbm.at[sbuf[q]], rows[p], gsem[p])

    def wait_gather(q, p):
      pltpu.make_async_copy(u_hbm.at[sbuf[q]], rows[p], gsem[p]).wait()

    def fire_scatter(q, p):
      pltpu.async_copy(rows[p], agg_sh.at[dbuf[q]], ssem[p], add=True)

    def drain_scatter(q, p):
      pltpu.make_async_copy(rows[p], agg_sh.at[dbuf[q]], ssem[p]).wait()

    load_idx(0, 0)
    load_idx(1, 1)
    fire_gather(0, 0)
    wait_gather(0, 0)
    fire_scatter(0, 0)
    fire_gather(1, 1)
    load_idx(2, 2)
    wait_gather(1, 1)
    fire_scatter(1, 1)
    drain_scatter(0, 0)
    fire_gather(2, 0)
    load_idx(3, 3)

    def step(gv, q, p):
      wait_gather(q, p)
      fire_scatter(q, p)
      drain_scatter((q - 1) % 4, 1 - p)
      fire_gather((q + 1) % 4, 1 - p)
      load_idx(gv + 2, (q + 2) % 4)

    def body(k, carry):
      g0v = 2 + 4 * k
      step(g0v, 2, 0)
      step(g0v + 1, 3, 1)
      step(g0v + 2, 0, 0)
      step(g0v + 3, 1, 1)
      return carry

    lax.fori_loop(0, (groups - 2) // 4, body, 0)
    wait_gather(2, 0)
    drain_scatter(1, 1)
    plsc.subcore_barrier()
    pltpu.sync_copy(agg_sh.at[pl.ds(s * rpt, rpt)],
                    out_hbm.at[c].at[pl.ds(s * rpt, rpt)])

  return edge_kernel


def _prep_body(degp_ref, x_ref, w_ref, dinv_ref, u_ref):
  d = degp_ref[0] + degp_ref[1] + 1.0
  dv = lax.rsqrt(d)
  dinv_ref[...] = dv
  u_ref[...] = dv * jnp.dot(x_ref[...], w_ref[...],
                            preferred_element_type=jnp.float32)


def _layer_body(agg_ref, u_ref, dinv_ref, b_ref, w_ref, un_ref):
  t = agg_ref[0] + agg_ref[1] + u_ref[...]
  o = jnp.maximum(dinv_ref[...] * t + b_ref[...], 0.0)
  un_ref[...] = dinv_ref[...] * jnp.dot(o, w_ref[...],
                                        preferred_element_type=jnp.float32)


def _make_final_body(nblk, nbp, npk_x):
  def _final_body(agg_ref, u_ref, dinv_ref, b_ref, batch_ref, wl_ref, bl_ref,
                  out_ref, acc_ref):
    i = pl.program_id(0)
    t = agg_ref[0] + agg_ref[1] + u_ref[...]
    o = jnp.maximum(dinv_ref[...] * t + b_ref[...], 0.0)
    rows = lax.broadcasted_iota(jnp.int32, (nbp, 1), 0) + i * nbp
    o = jnp.where(rows < npk_x, o, 0.0)
    ids = lax.broadcasted_iota(jnp.int32, (1, 64), 1)
    ones = jnp.ones((nbp, 16), jnp.float32)
    part = jnp.zeros((64, 32), jnp.float32)
    for k in range(8):
      oh = (batch_ref[:, k:k + 1] == ids).astype(jnp.float32)
      ext = jnp.concatenate([o[:, 16 * k:16 * k + 16], ones], axis=1)
      part = part + lax.dot_general(oh, ext, (((0,), (0,)), ((), ())),
                                    preferred_element_type=jnp.float32)

    @pl.when(i == 0)
    def _():
      acc_ref[...] = jnp.zeros_like(acc_ref)

    acc_ref[...] += part

    @pl.when(i == nblk - 1)
    def _():
      sums = acc_ref[:, :16]
      cnt = acc_ref[:, 16:17]
      pooled = sums / jnp.maximum(cnt, 1.0)
      out_ref[...] = jnp.dot(pooled, wl_ref[...],
                             preferred_element_type=jnp.float32) + bl_ref[...]

  return _final_body


def kernel(x, edge_index, batch, W1, b1, W2, b2, W3, b3, Wl, bl):
  n = x.shape[0]
  e = edge_index.shape[1]
  f32 = jnp.float32

  rpt = ((n + NS - 1) // NS + 7) // 8 * 8
  np_rows = NS * rpt
  npk = np_rows // 8
  npk_x = n * 16 // 128
  nblk = 4
  nbp = npk // nblk
  t_pair = (e + 16 * GSZ - 1) // (16 * GSZ)
  t_pair += (-t_pair) % 4
  g0 = 2 + 4 * max(1, int(round((t_pair * FAST_CORE_SHARE - 2) / 4)))
  g1 = t_pair - g0
  e_pad = 16 * GSZ * t_pair
  pad = e_pad - e
  e_arr = e_pad + 2 * GSZ

  src = edge_index[0]
  dst = edge_index[1]
  pad_src = jnp.zeros((e_arr - e,), jnp.int32)
  pad_dst = jnp.concatenate([
      n + (jnp.arange(pad, dtype=jnp.int32) % (np_rows - n)),
      jnp.zeros((2 * GSZ,), jnp.int32),
  ])
  src_p = jnp.concatenate([src, pad_src])
  dst_p = jnp.concatenate([dst, pad_dst])

  ones16 = jnp.ones((GSZ, 16), f32)
  zeros16 = jnp.zeros((rpt, 16), f32)

  x_p = x.reshape(npk_x, 128)
  batch_p = jnp.concatenate(
      [batch, jnp.full((np_rows - n,), 64, batch.dtype)]).reshape(npk, 8)
  eye8 = jnp.eye(8, dtype=f32)
  w1bd = jnp.kron(eye8, W1)
  w2bd = jnp.kron(eye8, W2)
  w3bd = jnp.kron(eye8, W3)
  b1t = jnp.tile(b1, 8).reshape(1, 128)
  b2t = jnp.tile(b2, 8).reshape(1, 128)
  b3t = jnp.tile(b3, 8).reshape(1, 128)

  deg_parts = _make_deg_kernel(t_pair // 2, rpt, np_rows)(
      dst_p, ones16, zeros16)
  deg_r = deg_parts.reshape(NC, npk, 128)
  edge_kernel = _make_edge_kernel(g0, g1, rpt, np_rows)

  row_spec = pl.BlockSpec((nbp, 128), lambda i: (i, 0))
  agg_spec = pl.BlockSpec((2, nbp, 128), lambda i: (0, i, 0))
  w_spec = pl.BlockSpec((128, 128), lambda i: (0, 0))
  b_spec = pl.BlockSpec((1, 128), lambda i: (0, 0))

  dinv, u = pl.pallas_call(
      _prep_body,
      grid=(nblk,),
      in_specs=[agg_spec, row_spec, w_spec],
      out_specs=[row_spec, row_spec],
      out_shape=[
          jax.ShapeDtypeStruct((npk_x, 128), f32),
          jax.ShapeDtypeStruct((npk_x, 128), f32),
      ],
  )(deg_r, x_p, w1bd)

  layer_call = pl.pallas_call(
      _layer_body,
      grid=(nblk,),
      in_specs=[agg_spec, row_spec, row_spec, b_spec, w_spec],
      out_specs=row_spec,
      out_shape=jax.ShapeDtypeStruct((npk_x, 128), f32),
  )

  agg = edge_kernel(src_p, dst_p, u.reshape(n, 16), zeros16)
  u = layer_call(agg.reshape(NC, npk, 128), u, dinv, b1t, w2bd)
  agg = edge_kernel(src_p, dst_p, u.reshape(n, 16), zeros16)
  u = layer_call(agg.reshape(NC, npk, 128), u, dinv, b2t, w3bd)
  agg = edge_kernel(src_p, dst_p, u.reshape(n, 16), zeros16)

  out = pl.pallas_call(
      _make_final_body(nblk, nbp, npk_x),
      grid=(nblk,),
      in_specs=[
          agg_spec, row_spec, row_spec, b_spec,
          pl.BlockSpec((nbp, 8), lambda i: (i, 0)),
          pl.BlockSpec((16, 2), lambda i: (0, 0)),
          pl.BlockSpec((1, 2), lambda i: (0, 0)),
      ],
      out_specs=pl.BlockSpec((64, 2), lambda i: (0, 0)),
      out_shape=jax.ShapeDtypeStruct((64, 2), f32),
      scratch_shapes=[pltpu.VMEM((64, 32), f32)],
  )(agg.reshape(NC, npk, 128), u, dinv, b3t, batch_p, Wl,
    bl.reshape(1, 2))

  return out


# speedup vs baseline: 1.0477x; 1.0477x over previous
import functools

import jax
import jax.numpy as jnp
from jax import lax
from jax.experimental import pallas as pl
from jax.experimental.pallas import tpu as pltpu
from jax.experimental.pallas import tpu_sc as plsc

NC = 2
NS = 16
GSZ = 512
FAST_CORE_SHARE = 0.555


def _mesh():
  return plsc.VectorSubcoreMesh(core_axis_name="c", subcore_axis_name="s")


def _make_deg_kernel(groups, rpt, np_rows):
  @functools.partial(
      pl.kernel,
      out_type=jax.ShapeDtypeStruct((NC, np_rows, 16), jnp.float32),
      mesh=_mesh(),
      compiler_params=pltpu.CompilerParams(use_tc_tiling_on_sc=False),
      scratch_types=[
          pltpu.VMEM((GSZ,), jnp.int32),
          pltpu.VMEM((GSZ,), jnp.int32),
          pltpu.VMEM((GSZ, 16), jnp.float32),
          pltpu.SemaphoreType.DMA,
          pltpu.SemaphoreType.DMA,
          pltpu.VMEM_SHARED((np_rows, 16), jnp.float32),
      ],
  )
  def deg_kernel(dst_hbm, ones_hbm, zeros_hbm, out_hbm,
                 dbuf0, dbuf1, ones_v, ssem0, ssem1, deg_sh):
    c = lax.axis_index("c")
    s = lax.axis_index("s")
    dbuf = (dbuf0, dbuf1)
    ssem = (ssem0, ssem1)
    pltpu.sync_copy(ones_hbm, ones_v)
    pltpu.sync_copy(zeros_hbm, deg_sh.at[pl.ds(s * rpt, rpt)])
    plsc.subcore_barrier()
    tb = (c * NS + s) * groups * GSZ

    def fire(gv, p):
      pltpu.sync_copy(dst_hbm.at[pl.ds(tb + gv * GSZ, GSZ)], dbuf[p])
      pltpu.async_copy(ones_v, deg_sh.at[dbuf[p]], ssem[p], add=True)

    def drain(p):
      pltpu.make_async_copy(ones_v, deg_sh.at[dbuf[p]], ssem[p]).wait()

    fire(0, 0)
    fire(1, 1)

    def body(k, carry):
      g = 2 + 2 * k
      drain(0)
      fire(g, 0)
      drain(1)
      fire(g + 1, 1)
      return carry

    lax.fori_loop(0, (groups - 2) // 2, body, 0)
    drain(0)
    drain(1)
    plsc.subcore_barrier()
    pltpu.sync_copy(deg_sh.at[pl.ds(s * rpt, rpt)],
                    out_hbm.at[c].at[pl.ds(s * rpt, rpt)])

  return deg_kernel


def _make_edge_kernel(g0, g1, rpt, np_rows):
  @functools.partial(
      pl.kernel,
      out_type=jax.ShapeDtypeStruct((NC, np_rows, 16), jnp.float32),
      mesh=_mesh(),
      compiler_params=pltpu.CompilerParams(use_tc_tiling_on_sc=False),
      scratch_types=[
          [pltpu.VMEM((GSZ,), jnp.int32)] * 4,
          [pltpu.VMEM((GSZ,), jnp.int32)] * 4,
          [pltpu.VMEM((GSZ, 16), jnp.float32)] * 2,
          [pltpu.SemaphoreType.DMA] * 2,
          [pltpu.SemaphoreType.DMA] * 2,
          pltpu.VMEM_SHARED((np_rows, 16), jnp.float32),
      ],
  )
  def edge_kernel(src_hbm, dst_hbm, u_hbm, zeros_hbm, out_hbm,
                  sbuf, dbuf, rows, gsem, ssem, agg_sh):
    c = lax.axis_index("c")
    s = lax.axis_index("s")
    pltpu.sync_copy(zeros_hbm, agg_sh.at[pl.ds(s * rpt, rpt)])
    plsc.subcore_barrier()
    groups = lax.select(c == 0, g0, g1)
    tb = lax.select(c == 0, s * g0, NS * g0 + s * g1) * GSZ

    def load_idx(gv, q):
      pltpu.sync_copy(src_hbm.at[pl.ds(tb + gv * GSZ, GSZ)], sbuf[q])
      pltpu.sync_copy(dst_hbm.at[pl.ds(tb + gv * GSZ, GSZ)], dbuf[q])

    def fire_gather(q, p):
      pltpu.async_copy(u_hbm.at[sbuf[q]], rows[p], gsem[p])

    def wait_gather(q, p):
      pltpu.make_async_copy(u_hbm.at[sbuf[q]], rows[p], gsem[p]).wait()

    def fire_scatter(q, p):
      pltpu.async_copy(rows[p], agg_sh.at[dbuf[q]], ssem[p], add=True)

    def drain_scatter(q, p):
      pltpu.make_async_copy(rows[p], agg_sh.at[dbuf[q]], ssem[p]).wait()

    load_idx(0, 0)
    load_idx(1, 1)
    fire_gather(0, 0)
    wait_gather(0, 0)
    fire_scatter(0, 0)
    fire_gather(1, 1)
    load_idx(2, 2)
    wait_gather(1, 1)
    fire_scatter(1, 1)
    drain_scatter(0, 0)
    fire_gather(2, 0)
    load_idx(3, 3)

    def step(gv, q, p):
      wait_gather(q, p)
      fire_scatter(q, p)
      drain_scatter((q - 1) % 4, 1 - p)
      fire_gather((q + 1) % 4, 1 - p)
      load_idx(gv + 2, (q + 2) % 4)

    def body(k, carry):
      g0v = 2 + 4 * k
      step(g0v, 2, 0)
      step(g0v + 1, 3, 1)
      step(g0v + 2, 0, 0)
      step(g0v + 3, 1, 1)
      return carry

    lax.fori_loop(0, (groups - 2) // 4, body, 0)
    wait_gather(2, 0)
    drain_scatter(1, 1)
    plsc.subcore_barrier()
    pltpu.sync_copy(agg_sh.at[pl.ds(s * rpt, rpt)],
                    out_hbm.at[c].at[pl.ds(s * rpt, rpt)])

  return edge_kernel


def _prep_body(degp_ref, x_ref, w_ref, dinv_ref, u_ref):
  d = degp_ref[0] + degp_ref[1] + 1.0
  dv = lax.rsqrt(d)
  dinv_ref[...] = dv
  u_ref[...] = dv * jnp.dot(x_ref[...], w_ref[...],
                            preferred_element_type=jnp.float32)


def _layer_body(agg_ref, u_ref, dinv_ref, b_ref, w_ref, un_ref):
  t = agg_ref[0] + agg_ref[1] + u_ref[...]
  o = jnp.maximum(dinv_ref[...] * t + b_ref[...], 0.0)
  un_ref[...] = dinv_ref[...] * jnp.dot(o, w_ref[...],
                                        preferred_element_type=jnp.float32)


def _make_final_body(nblk, nbp, npk_x):
  def _final_body(agg_ref, u_ref, dinv_ref, b_ref, batch_ref, wl_ref, bl_ref,
                  out_ref, acc_ref):
    i = pl.program_id(0)
    t = agg_ref[0] + agg_ref[1] + u_ref[...]
    o = jnp.maximum(dinv_ref[...] * t + b_ref[...], 0.0)
    rows = lax.broadcasted_iota(jnp.int32, (nbp, 1), 0) + i * nbp
    o = jnp.where(rows < npk_x, o, 0.0)
    ids = lax.broadcasted_iota(jnp.int32, (1, 64), 1)
    ones = jnp.ones((nbp, 16), jnp.float32)
    part = jnp.zeros((64, 32), jnp.float32)
    for k in range(8):
      oh = (batch_ref[:, k:k + 1] == ids).astype(jnp.float32)
      ext = jnp.concatenate([o[:, 16 * k:16 * k + 16], ones], axis=1)
      part = part + lax.dot_general(oh, ext, (((0,), (0,)), ((), ())),
                                    preferred_element_type=jnp.float32)

    @pl.when(i == 0)
    def _():
      acc_ref[...] = jnp.zeros_like(acc_ref)

    acc_ref[...] += part

    @pl.when(i == nblk - 1)
    def _():
      sums = acc_ref[:, :16]
      cnt = acc_ref[:, 16:17]
      pooled = sums / jnp.maximum(cnt, 1.0)
      out_ref[...] = jnp.dot(pooled, wl_ref[...],
                             preferred_element_type=jnp.float32) + bl_ref[...]

  return _final_body


def kernel(x, edge_index, batch, W1, b1, W2, b2, W3, b3, Wl, bl):
  n = x.shape[0]
  e = edge_index.shape[1]
  f32 = jnp.float32

  rpt = ((n + NS - 1) // NS + 7) // 8 * 8
  np_rows = NS * rpt
  npk = np_rows // 8
  npk_x = n * 16 // 128
  nblk = 4
  nbp = npk // nblk
  t_pair = (e + 16 * GSZ - 1) // (16 * GSZ)
  t_pair += (-t_pair) % 4
  g0 = 2 + 4 * max(1, int(round((t_pair * FAST_CORE_SHARE - 2) / 4)))
  g1 = t_pair - g0
  e_pad = 16 * GSZ * t_pair
  pad = e_pad - e
  e_arr = e_pad + 2 * GSZ

  src = edge_index[0]
  dst = edge_index[1]
  pad_src = jnp.zeros((e_arr - e,), jnp.int32)
  pad_dst = jnp.concatenate([
      n + (jnp.arange(pad, dtype=jnp.int32) % (np_rows - n)),
      jnp.zeros((2 * GSZ,), jnp.int32),
  ])
  src_p = jnp.concatenate([src, pad_src])
  dst_p = jnp.concatenate([dst, pad_dst])

  ones16 = jnp.ones((GSZ, 16), f32)
  zeros16 = jnp.zeros((rpt, 16), f32)

  x_p = x.reshape(npk_x, 128)
  batch_p = jnp.concatenate(
      [batch, jnp.full((np_rows - n,), 64, batch.dtype)]).reshape(npk, 8)
  eye8 = jnp.eye(8, dtype=f32)
  w1bd = jnp.kron(eye8, W1)
  w2bd = jnp.kron(eye8, W2)
  w3bd = jnp.kron(eye8, W3)
  b1t = jnp.tile(b1, 8).reshape(1, 128)
  b2t = jnp.tile(b2, 8).reshape(1, 128)
  b3t = jnp.tile(b3, 8).reshape(1, 128)

  deg_parts = _make_deg_kernel(t_pair // 2, rpt, np_rows)(
      dst_p, ones16, zeros16)
  deg_r = deg_parts.reshape(NC, npk, 128)
  edge_kernel = _make_edge_kernel(g0, g1, rpt, np_rows)

  row_spec = pl.BlockSpec((nbp, 128), lambda i: (i, 0))
  agg_spec = pl.BlockSpec((2, nbp, 128), lambda i: (0, i, 0))
  w_spec = pl.BlockSpec((128, 128), lambda i: (0, 0))
  b_spec = pl.BlockSpec((1, 128), lambda i: (0, 0))

  dinv, u = pl.pallas_call(
      _prep_body,
      grid=(nblk,),
      in_specs=[agg_spec, row_spec, w_spec],
      out_specs=[row_spec, row_spec],
      out_shape=[
          jax.ShapeDtypeStruct((npk_x, 128), f32),
          jax.ShapeDtypeStruct((npk_x, 128), f32),
      ],
  )(deg_r, x_p, w1bd)

  layer_call = pl.pallas_call(
      _layer_body,
      grid=(nblk,),
      in_specs=[agg_spec, row_spec, row_spec, b_spec, w_spec],
      out_specs=row_spec,
      out_shape=jax.ShapeDtypeStruct((npk_x, 128), f32),
  )

  agg = edge_kernel(src_p, dst_p, u.reshape(n, 16), zeros16)
  u = layer_call(agg.reshape(NC, npk, 128), u, dinv, b1t, w2bd)
  agg = edge_kernel(src_p, dst_p, u.reshape(n, 16), zeros16)
  u = layer_call(agg.reshape(NC, npk, 128), u, dinv, b2t, w3bd)
  agg = edge_kernel(src_p, dst_p, u.reshape(n, 16), zeros16)

  out = pl.pallas_call(
      _make_final_body(nblk, nbp, npk_x),
      grid=(nblk,),
      in_specs=[
          agg_spec, row_spec, row_spec, b_spec,
          pl.BlockSpec((nbp, 8), lambda i: (i, 0)),
          pl.BlockSpec((16, 2), lambda i: (0, 0)),
          pl.BlockSpec((1, 2), lambda i: (0, 0)),
      ],
      out_specs=pl.BlockSpec((64, 2), lambda i: (0, 0)),
      out_shape=jax.ShapeDtypeStruct((64, 2), f32),
      scratch_shapes=[pltpu.VMEM((64, 32), f32)],
  )(agg.reshape(NC, npk, 128), u, dinv, b3t, batch_p, Wl,
    bl.reshape(1, 2))

  return out
